# baseline (speedup 1.0000x reference)
import functools

import jax
import jax.numpy as jnp
from jax import lax
from jax.experimental import pallas as pl
from jax.experimental.pallas import tpu as pltpu
from jax.experimental.pallas import tpu_sc as plsc

_ROW = 2048


_BLOCK_ROWS = 1024


def _tc_softmax_block(f_ref, od_ref, out_ref):
    f = f_ref[...]
    od = od_ref[...]
    mask = jnp.logical_and(od != 0, f != 0.0)
    neg_inf = jnp.float32(-jnp.inf)
    v = jnp.where(mask, f, neg_inf)
    mx = jnp.max(v, axis=-1, keepdims=True)
    mx = jnp.where(jnp.isfinite(mx), mx, jnp.zeros_like(mx))
    e = jnp.where(mask, jnp.exp(f - mx), 0.0)
    s = jnp.sum(e, axis=-1, keepdims=True)
    s = jnp.where(s == 0.0, jnp.ones_like(s), s)
    out_ref[...] = e / s


def _tc_softmax(f2, od2):
    rows = f2.shape[0]
    return pl.pallas_call(
        _tc_softmax_block,
        grid=(rows // _BLOCK_ROWS,),
        in_specs=[
            pl.BlockSpec((_BLOCK_ROWS, _ROW), lambda i: (i, 0)),
            pl.BlockSpec((_BLOCK_ROWS, _ROW), lambda i: (i, 0)),
        ],
        out_specs=pl.BlockSpec((_BLOCK_ROWS, _ROW), lambda i: (i, 0)),
        out_shape=jax.ShapeDtypeStruct((rows, _ROW), f2.dtype),
        compiler_params=pltpu.CompilerParams(
            dimension_semantics=("arbitrary",),
        ),
    )(f2, od2)


_NC, _NS, _L = 2, 16, 16
_NW = _NC * _NS
_CHUNK = 8
_NV = _ROW // _L


def _sc_body(f_hbm, od_hbm, out_hbm, f_v, od_v):
    rows = f_hbm.shape[0]
    rpw = rows // _NW
    nchunk = rpw // _CHUNK
    wid = lax.axis_index("s") * _NC + lax.axis_index("c")
    base = wid * rpw
    neg_inf = jnp.float32(-jnp.inf)

    def chunk_body(c, carry):
        row0 = base + c * _CHUNK
        pltpu.sync_copy(f_hbm.at[pl.ds(row0, _CHUNK)], f_v)
        pltpu.sync_copy(od_hbm.at[pl.ds(row0, _CHUNK)], od_v)

        def row_body(r, carry2):
            def p1(i, mx):
                f = f_v[r, pl.ds(i * _L, _L)]
                od = od_v[r, pl.ds(i * _L, _L)]
                m = jnp.logical_and(od != 0, f != 0.0)
                return jnp.maximum(mx, jnp.where(m, f, neg_inf))

            mxv = lax.fori_loop(0, _NV, p1, jnp.full((_L,), neg_inf, jnp.float32))
            mx = mxv[0]
            for j in range(1, _L):
                mx = jnp.maximum(mx, mxv[j])
            mx = jnp.where(mx == neg_inf, jnp.float32(0.0), mx)

            def p2(i, s):
                f = f_v[r, pl.ds(i * _L, _L)]
                od = od_v[r, pl.ds(i * _L, _L)]
                m = jnp.logical_and(od != 0, f != 0.0)
                e = jnp.where(m, jnp.exp(f - mx), jnp.float32(0.0))
                f_v[r, pl.ds(i * _L, _L)] = e
                return s + e

            sv = lax.fori_loop(0, _NV, p2, jnp.zeros((_L,), jnp.float32))
            s = sv[0]
            for j in range(1, _L):
                s = s + sv[j]
            s = jnp.where(s == 0.0, jnp.float32(1.0), s)
            rcp = jnp.ones((_L,), jnp.float32) / jnp.broadcast_to(s, (_L,))

            def p3(i, carry3):
                f_v[r, pl.ds(i * _L, _L)] = f_v[r, pl.ds(i * _L, _L)] * rcp
                return carry3

            lax.fori_loop(0, _NV, p3, 0)
            return carry2

        lax.fori_loop(0, _CHUNK, row_body, 0)
        pltpu.sync_copy(f_v, out_hbm.at[pl.ds(row0, _CHUNK)])
        return carry

    lax.fori_loop(0, nchunk, chunk_body, 0)


def _sc_softmax(f2, od2):
    rows = f2.shape[0]
    mesh = plsc.VectorSubcoreMesh(core_axis_name="c", subcore_axis_name="s")
    k = functools.partial(
        pl.kernel,
        mesh=mesh,
        out_type=jax.ShapeDtypeStruct((rows, _ROW), jnp.float32),
        scratch_types=[
            pltpu.VMEM((_CHUNK, _ROW), jnp.float32),
            pltpu.VMEM((_CHUNK, _ROW), jnp.int32),
        ],
    )(_sc_body)
    return k(f2, od2)


def kernel(features, OD):
    shape = features.shape
    rows = 1
    for d in shape[:-1]:
        rows *= d
    f2 = features.reshape(rows, shape[-1])
    od2 = OD.reshape(rows, shape[-1])
    out = _sc_softmax(f2, od2)
    return out.reshape(shape)


# speedup vs baseline: 2.3640x; 2.3640x over previous
import functools

import jax
import jax.numpy as jnp
from jax import lax
from jax.experimental import pallas as pl
from jax.experimental.pallas import tpu as pltpu
from jax.experimental.pallas import tpu_sc as plsc

_ROW = 2048


_BLOCK_ROWS = 1024


def _tc_softmax_block(f_ref, od_ref, out_ref):
    f = f_ref[...]
    od = od_ref[...]
    mask = jnp.logical_and(od != 0, f != 0.0)
    neg_inf = jnp.float32(-jnp.inf)
    v = jnp.where(mask, f, neg_inf)
    mx = jnp.max(v, axis=-1, keepdims=True)
    mx = jnp.where(jnp.isfinite(mx), mx, jnp.zeros_like(mx))
    e = jnp.where(mask, jnp.exp(f - mx), 0.0)
    s = jnp.sum(e, axis=-1, keepdims=True)
    s = jnp.where(s == 0.0, jnp.ones_like(s), s)
    out_ref[...] = e / s


def _tc_softmax(f2, od2):
    rows = f2.shape[0]
    return pl.pallas_call(
        _tc_softmax_block,
        grid=(rows // _BLOCK_ROWS,),
        in_specs=[
            pl.BlockSpec((_BLOCK_ROWS, _ROW), lambda i: (i, 0)),
            pl.BlockSpec((_BLOCK_ROWS, _ROW), lambda i: (i, 0)),
        ],
        out_specs=pl.BlockSpec((_BLOCK_ROWS, _ROW), lambda i: (i, 0)),
        out_shape=jax.ShapeDtypeStruct((rows, _ROW), f2.dtype),
        compiler_params=pltpu.CompilerParams(
            dimension_semantics=("arbitrary",),
        ),
    )(f2, od2)


_NC, _NS, _L = 2, 16, 16
_NW = _NC * _NS
_CHUNK = 16
_NV = _ROW // _L


def _sc_body(f_hbm, od_hbm, out_hbm, f_v, od_v):
    rows = f_hbm.shape[0]
    rpw = rows // _NW
    nchunk = rpw // _CHUNK
    wid = lax.axis_index("s") * _NC + lax.axis_index("c")
    base = wid * rpw
    neg_inf = jnp.float32(-jnp.inf)

    def chunk_body(c, carry):
        row0 = base + c * _CHUNK
        pltpu.sync_copy(f_hbm.at[pl.ds(row0, _CHUNK)], f_v)
        pltpu.sync_copy(od_hbm.at[pl.ds(row0, _CHUNK)], od_v)

        def row_body(r, carry2):
            def p1(i, mx):
                f = f_v[r, pl.ds(i * _L, _L)]
                od = od_v[r, pl.ds(i * _L, _L)]
                m = jnp.logical_and(od != 0, f != 0.0)
                v = jnp.where(m, f, neg_inf)
                f_v[r, pl.ds(i * _L, _L)] = v
                return jnp.maximum(mx, v)

            mxv = lax.fori_loop(
                0, _NV, p1, jnp.full((_L,), neg_inf, jnp.float32), unroll=8
            )
            mx = mxv[0]
            for j in range(1, _L):
                mx = jnp.maximum(mx, mxv[j])
            mx = jnp.where(mx == neg_inf, jnp.float32(0.0), mx)

            def p2(i, s):
                v = f_v[r, pl.ds(i * _L, _L)]
                e = jnp.exp(v - mx)
                f_v[r, pl.ds(i * _L, _L)] = e
                return s + e

            sv = lax.fori_loop(
                0, _NV, p2, jnp.zeros((_L,), jnp.float32), unroll=8
            )
            s = sv[0]
            for j in range(1, _L):
                s = s + sv[j]
            s = jnp.where(s == 0.0, jnp.float32(1.0), s)
            rcp = jnp.ones((_L,), jnp.float32) / jnp.broadcast_to(s, (_L,))

            def p3(i, carry3):
                f_v[r, pl.ds(i * _L, _L)] = f_v[r, pl.ds(i * _L, _L)] * rcp
                return carry3

            lax.fori_loop(0, _NV, p3, 0, unroll=8)
            return carry2

        lax.fori_loop(0, _CHUNK, row_body, 0)
        pltpu.sync_copy(f_v, out_hbm.at[pl.ds(row0, _CHUNK)])
        return carry

    lax.fori_loop(0, nchunk, chunk_body, 0)


def _sc_softmax(f2, od2):
    rows = f2.shape[0]
    mesh = plsc.VectorSubcoreMesh(core_axis_name="c", subcore_axis_name="s")
    k = functools.partial(
        pl.kernel,
        mesh=mesh,
        out_type=jax.ShapeDtypeStruct((rows, _ROW), jnp.float32),
        scratch_types=[
            pltpu.VMEM((_CHUNK, _ROW), jnp.float32),
            pltpu.VMEM((_CHUNK, _ROW), jnp.int32),
        ],
    )(_sc_body)
    return k(f2, od2)


def kernel(features, OD):
    shape = features.shape
    rows = 1
    for d in shape[:-1]:
        rows *= d
    f2 = features.reshape(rows, shape[-1])
    od2 = OD.reshape(rows, shape[-1])
    out = _sc_softmax(f2, od2)
    return out.reshape(shape)


# speedup vs baseline: 13.0947x; 5.5392x over previous
import functools

import jax
import jax.numpy as jnp
from jax import lax
from jax.experimental import pallas as pl
from jax.experimental.pallas import tpu as pltpu
from jax.experimental.pallas import tpu_sc as plsc

_ROW = 2048


_BLOCK_ROWS = 768


def _tc_softmax_block(f_ref, od_ref, out_ref):
    f = f_ref[...]
    od = od_ref[...]
    mask = jnp.logical_and(od != 0, f != 0.0)
    neg_inf = jnp.float32(-jnp.inf)
    v = jnp.where(mask, f, neg_inf)
    mx = jnp.max(v, axis=-1, keepdims=True)
    mx = jnp.where(jnp.isfinite(mx), mx, jnp.zeros_like(mx))
    e = jnp.where(mask, jnp.exp(f - mx), 0.0)
    s = jnp.sum(e, axis=-1, keepdims=True)
    s = jnp.where(s == 0.0, jnp.ones_like(s), s)
    out_ref[...] = e / s


def _tc_softmax(f2, od2):
    rows = f2.shape[0]
    return pl.pallas_call(
        _tc_softmax_block,
        grid=(rows // _BLOCK_ROWS,),
        in_specs=[
            pl.BlockSpec((_BLOCK_ROWS, _ROW), lambda i: (i, 0)),
            pl.BlockSpec((_BLOCK_ROWS, _ROW), lambda i: (i, 0)),
        ],
        out_specs=pl.BlockSpec((_BLOCK_ROWS, _ROW), lambda i: (i, 0)),
        out_shape=jax.ShapeDtypeStruct((rows, _ROW), f2.dtype),
        compiler_params=pltpu.CompilerParams(
            dimension_semantics=("arbitrary",),
        ),
    )(f2, od2)


_NC, _NS, _L = 2, 16, 16
_NW = _NC * _NS
_CHUNK = 16
_NV = _ROW // _L


def _sc_body(f_hbm, od_hbm, out_hbm, f_v, od_v):
    rows = f_hbm.shape[0]
    rpw = rows // _NW
    nchunk = rpw // _CHUNK
    wid = lax.axis_index("s") * _NC + lax.axis_index("c")
    base = wid * rpw
    neg_inf = jnp.float32(-jnp.inf)

    def chunk_body(c, carry):
        row0 = base + c * _CHUNK
        pltpu.sync_copy(f_hbm.at[pl.ds(row0, _CHUNK)], f_v)
        pltpu.sync_copy(od_hbm.at[pl.ds(row0, _CHUNK)], od_v)

        def row_body(r, carry2):
            def p1(i, mx):
                f = f_v[r, pl.ds(i * _L, _L)]
                od = od_v[r, pl.ds(i * _L, _L)]
                m = jnp.logical_and(od != 0, f != 0.0)
                v = jnp.where(m, f, neg_inf)
                f_v[r, pl.ds(i * _L, _L)] = v
                return jnp.maximum(mx, v)

            mxv = lax.fori_loop(
                0, _NV, p1, jnp.full((_L,), neg_inf, jnp.float32), unroll=8
            )
            mx = mxv[0]
            for j in range(1, _L):
                mx = jnp.maximum(mx, mxv[j])
            mx = jnp.where(mx == neg_inf, jnp.float32(0.0), mx)

            def p2(i, s):
                v = f_v[r, pl.ds(i * _L, _L)]
                e = jnp.exp(v - mx)
                f_v[r, pl.ds(i * _L, _L)] = e
                return s + e

            sv = lax.fori_loop(
                0, _NV, p2, jnp.zeros((_L,), jnp.float32), unroll=8
            )
            s = sv[0]
            for j in range(1, _L):
                s = s + sv[j]
            s = jnp.where(s == 0.0, jnp.float32(1.0), s)
            rcp = jnp.ones((_L,), jnp.float32) / jnp.broadcast_to(s, (_L,))

            def p3(i, carry3):
                f_v[r, pl.ds(i * _L, _L)] = f_v[r, pl.ds(i * _L, _L)] * rcp
                return carry3

            lax.fori_loop(0, _NV, p3, 0, unroll=8)
            return carry2

        lax.fori_loop(0, _CHUNK, row_body, 0)
        pltpu.sync_copy(f_v, out_hbm.at[pl.ds(row0, _CHUNK)])
        return carry

    lax.fori_loop(0, nchunk, chunk_body, 0)


def _sc_softmax(f2, od2):
    rows = f2.shape[0]
    mesh = plsc.VectorSubcoreMesh(core_axis_name="c", subcore_axis_name="s")
    k = functools.partial(
        pl.kernel,
        mesh=mesh,
        out_type=jax.ShapeDtypeStruct((rows, _ROW), jnp.float32),
        scratch_types=[
            pltpu.VMEM((_CHUNK, _ROW), jnp.float32),
            pltpu.VMEM((_CHUNK, _ROW), jnp.int32),
        ],
    )(_sc_body)
    return k(f2, od2)


def kernel(features, OD):
    shape = features.shape
    rows = 1
    for d in shape[:-1]:
        rows *= d
    f2 = features.reshape(rows, shape[-1])
    od2 = OD.reshape(rows, shape[-1])
    out = _tc_softmax(f2, od2)
    return out.reshape(shape)
